# baseline (speedup 1.0000x reference)
import functools

import jax
import jax.numpy as jnp
from jax import lax
from jax.experimental import pallas as pl
from jax.experimental.pallas import tpu as pltpu
from jax.experimental.pallas import tpu_sc as plsc

NUM_EMBEDDINGS = 1000000
EMBEDDING_DIM = 64
BATCH = 16384
HIST_LEN = 50

NC = 2
NS = 16
NW = NC * NS

G = 4
B_PER_W = BATCH // NW
STEPS = B_PER_W // G
ROWS = G * HIST_LEN

PACK_BLK = 1000


def _pack_body(w_ref, out_ref):
    x = w_ref[...]
    out_ref[:, :EMBEDDING_DIM] = x
    out_ref[:, EMBEDDING_DIM:] = x


def _pack(weight):
    return pl.pallas_call(
        _pack_body,
        grid=(NUM_EMBEDDINGS // PACK_BLK,),
        in_specs=[pl.BlockSpec((PACK_BLK, EMBEDDING_DIM), lambda i: (i, 0))],
        out_specs=pl.BlockSpec((PACK_BLK, 2 * EMBEDDING_DIM), lambda i: (i, 0)),
        out_shape=jax.ShapeDtypeStruct(
            (NUM_EMBEDDINGS, 2 * EMBEDDING_DIM), jnp.float32),
    )(weight)


def _gather_body(tok_hbm, w128_hbm, out_hbm, idx_v, rows128_v, rows_v, sem):
    wid = lax.axis_index("s") * NC + lax.axis_index("c")
    b_base = wid * B_PER_W

    def step(t, carry):
        b0 = b_base + t * G
        pltpu.sync_copy(tok_hbm.at[pl.ds(b0, G)], idx_v)
        copies = []
        for g in range(G):
            copies.append(pltpu.async_copy(
                w128_hbm.at[idx_v.at[g]],
                rows128_v.at[pl.ds(g * HIST_LEN, HIST_LEN)],
                sem))
        for c in copies:
            c.wait()

        def extract(k, carry2):
            for j in range(EMBEDDING_DIM // 16):
                rows_v[k, pl.ds(16 * j, 16)] = rows128_v[k, pl.ds(16 * j, 16)]
            return carry2

        lax.fori_loop(0, ROWS, extract, 0)
        for g in range(G):
            pltpu.sync_copy(
                rows_v.at[pl.ds(g * HIST_LEN, HIST_LEN)],
                out_hbm.at[b0 + g])
        return carry

    lax.fori_loop(0, STEPS, step, 0)


def _gather(token_ids, w128):
    mesh = plsc.VectorSubcoreMesh(core_axis_name="c", subcore_axis_name="s")
    f = pl.kernel(
        _gather_body,
        out_type=jax.ShapeDtypeStruct((BATCH, HIST_LEN, EMBEDDING_DIM),
                                      jnp.float32),
        mesh=mesh,
        scratch_types=[
            pltpu.VMEM((G, HIST_LEN), jnp.int32),
            pltpu.VMEM((ROWS, 2 * EMBEDDING_DIM), jnp.float32),
            pltpu.VMEM((ROWS, EMBEDDING_DIM), jnp.float32),
            pltpu.SemaphoreType.DMA,
        ],
    )
    return f(token_ids, w128)


@jax.jit
def _embed(token_ids, weight):
    return _gather(token_ids.astype(jnp.int32), _pack(weight))


def kernel(token_ids, weight):
    return _embed(token_ids, weight)


# speedup vs baseline: 1.0095x; 1.0095x over previous
import jax
import jax.numpy as jnp
from jax import lax
from jax.experimental import pallas as pl
from jax.experimental.pallas import tpu as pltpu
from jax.experimental.pallas import tpu_sc as plsc

NUM_EMBEDDINGS = 1000000
EMBEDDING_DIM = 64
BATCH = 16384
HIST_LEN = 50

NC = 2
NS = 16
NW = NC * NS

L = 16
BLK = 128
NPAIR = NUM_EMBEDDINGS // 2
COLS = NUM_EMBEDDINGS // BLK
PACK_K = 7808 // NW
PACK_PAIRS = PACK_K // 2
EXTRA_COLS = COLS - PACK_K * NW
TAIL_START = COLS * BLK
TAIL_N = NUM_EMBEDDINGS - TAIL_START
UNITS_PER_W = HIST_LEN * (BATCH // BLK) // NW
GATHER_PAIRS = UNITS_PER_W // 2


def _iota16():
    return lax.iota(jnp.int32, L)


def _transpose_block(s_ref, d_ref, nrows):
    for m in range(BLK // L):
        col_vec = _iota16() + (m * L)
        c_vec = lax.bitwise_and(col_vec, 63)
        k_vec = lax.shift_right_logical(col_vec, 6)
        for p0 in range(0, nrows, 8):
            vals = [plsc.load_gather(s_ref, [c_vec, k_vec + (2 * p)])
                    for p in range(p0, p0 + 8)]
            for i, p in enumerate(range(p0, p0 + 8)):
                d_ref[p, pl.ds(m * L, L)] = vals[i]


def _pack_body(wt_hbm, w2_hbm, s0, s1, d0, d1, st_v, dt_v,
               sin0, sin1, so0, so1):
    wid = lax.axis_index("s") * NC + lax.axis_index("c")
    ub = wid * PACK_K

    def in_copy(u, s_ref, sem):
        return pltpu.make_async_copy(
            wt_hbm.at[:, pl.ds(u * BLK, BLK)], s_ref, sem)

    def out_copy(u, d_ref, sem):
        return pltpu.make_async_copy(
            d_ref, w2_hbm.at[pl.ds(u * (BLK // 2), BLK // 2)], sem)

    in_copy(ub, s0, sin0).start()

    def itr(k, carry):
        u0 = ub + 2 * k
        in_copy(u0 + 1, s1, sin1).start()
        in_copy(u0, s0, sin0).wait()

        @pl.when(k > 0)
        def _():
            out_copy(u0 - 2, d0, so0).wait()
        _transpose_block(s0, d0, BLK // 2)
        out_copy(u0, d0, so0).start()

        @pl.when(k < PACK_PAIRS - 1)
        def _():
            in_copy(u0 + 2, s0, sin0).start()
        in_copy(u0 + 1, s1, sin1).wait()

        @pl.when(k > 0)
        def _():
            out_copy(u0 - 1, d1, so1).wait()
        _transpose_block(s1, d1, BLK // 2)
        out_copy(u0 + 1, d1, so1).start()
        return carry

    lax.fori_loop(0, PACK_PAIRS, itr, 0)
    out_copy(ub + PACK_K - 2, d0, so0).wait()
    out_copy(ub + PACK_K - 1, d1, so1).wait()

    @pl.when(wid < EXTRA_COLS)
    def _():
        u = PACK_K * NW + wid
        in_copy(u, s0, sin0).start()
        in_copy(u, s0, sin0).wait()
        _transpose_block(s0, d0, BLK // 2)
        out_copy(u, d0, so0).start()
        out_copy(u, d0, so0).wait()

    @pl.when(wid == 0)
    def _():
        pltpu.sync_copy(wt_hbm.at[:, pl.ds(TAIL_START, TAIL_N)], st_v)
        for m in range(BLK // L):
            col_vec = _iota16() + (m * L)
            c_vec = lax.bitwise_and(col_vec, 63)
            k_vec = lax.shift_right_logical(col_vec, 6)
            for p0 in range(0, TAIL_N // 2, 8):
                vals = [plsc.load_gather(st_v, [c_vec, k_vec + (2 * p)])
                        for p in range(p0, p0 + 8)]
                for i, p in enumerate(range(p0, p0 + 8)):
                    dt_v[p, pl.ds(m * L, L)] = vals[i]
        pltpu.sync_copy(dt_v, w2_hbm.at[pl.ds(NPAIR - TAIL_N // 2,
                                              TAIL_N // 2)])


def _pack(wt):
    mesh = plsc.VectorSubcoreMesh(core_axis_name="c", subcore_axis_name="s")
    f = pl.kernel(
        _pack_body,
        out_type=jax.ShapeDtypeStruct((NPAIR, BLK), jnp.float32),
        mesh=mesh,
        compiler_params=pltpu.CompilerParams(needs_layout_passes=False),
        scratch_types=[
            pltpu.VMEM((EMBEDDING_DIM, BLK), jnp.float32),
            pltpu.VMEM((EMBEDDING_DIM, BLK), jnp.float32),
            pltpu.VMEM((BLK // 2, BLK), jnp.float32),
            pltpu.VMEM((BLK // 2, BLK), jnp.float32),
            pltpu.VMEM((EMBEDDING_DIM, TAIL_N), jnp.float32),
            pltpu.VMEM((TAIL_N // 2, BLK), jnp.float32),
            pltpu.SemaphoreType.DMA,
            pltpu.SemaphoreType.DMA,
            pltpu.SemaphoreType.DMA,
            pltpu.SemaphoreType.DMA,
        ],
    )
    return f(wt)


def _gather_body(comb_hbm, w2_hbm, out_hbm, i0, i1, g0, g1, t0, t1,
                 sin0, sin1, sg0, sg1, so0, so1):
    wid = lax.axis_index("s") * NC + lax.axis_index("c")
    ub = wid * UNITS_PER_W

    def hu(uid):
        return lax.shift_right_logical(uid, 7), lax.bitwise_and(uid, 127)

    def in_copy(uid, i_ref, sem):
        h, u = hu(uid)
        return pltpu.make_async_copy(
            comb_hbm.at[h, :, pl.ds(u * BLK, BLK)], i_ref, sem)

    def g_copy(i_ref, g_ref, sem):
        return pltpu.make_async_copy(w2_hbm.at[i_ref.at[0]], g_ref, sem)

    def out_copy(uid, t_ref, sem):
        h, u = hu(uid)
        return pltpu.make_async_copy(
            t_ref, out_hbm.at[h, :, pl.ds(u * BLK, BLK)], sem)

    def transpose(i_ref, g_ref, t_ref):
        for m in range(BLK // L):
            row_vec = _iota16() + (m * L)
            b64 = i_ref[1, pl.ds(m * L, L)]
            for c0 in range(0, EMBEDDING_DIM, 8):
                vals = [plsc.load_gather(g_ref, [row_vec, b64 + c])
                        for c in range(c0, c0 + 8)]
                for i, c in enumerate(range(c0, c0 + 8)):
                    t_ref[c, pl.ds(m * L, L)] = vals[i]

    in_copy(ub, i0, sin0).start()
    in_copy(ub, i0, sin0).wait()
    g_copy(i0, g0, sg0).start()

    def itr(k, carry):
        u0 = ub + 2 * k
        in_copy(u0 + 1, i1, sin1).start()
        g_copy(i0, g0, sg0).wait()

        @pl.when(k > 0)
        def _():
            out_copy(u0 - 2, t0, so0).wait()
        transpose(i0, g0, t0)
        out_copy(u0, t0, so0).start()

        in_copy(u0 + 1, i1, sin1).wait()
        g_copy(i1, g1, sg1).start()

        @pl.when(k < GATHER_PAIRS - 1)
        def _():
            in_copy(u0 + 2, i0, sin0).start()
        g_copy(i1, g1, sg1).wait()

        @pl.when(k > 0)
        def _():
            out_copy(u0 - 1, t1, so1).wait()
        transpose(i1, g1, t1)
        out_copy(u0 + 1, t1, so1).start()

        @pl.when(k < GATHER_PAIRS - 1)
        def _():
            in_copy(u0 + 2, i0, sin0).wait()
            g_copy(i0, g0, sg0).start()
        return carry

    lax.fori_loop(0, GATHER_PAIRS, itr, 0)
    out_copy(ub + UNITS_PER_W - 2, t0, so0).wait()
    out_copy(ub + UNITS_PER_W - 1, t1, so1).wait()


def _gather(comb, w2):
    mesh = plsc.VectorSubcoreMesh(core_axis_name="c", subcore_axis_name="s")
    f = pl.kernel(
        _gather_body,
        out_type=jax.ShapeDtypeStruct((HIST_LEN, EMBEDDING_DIM, BATCH),
                                      jnp.float32),
        mesh=mesh,
        compiler_params=pltpu.CompilerParams(needs_layout_passes=False),
        scratch_types=[
            pltpu.VMEM((2, BLK), jnp.int32),
            pltpu.VMEM((2, BLK), jnp.int32),
            pltpu.VMEM((BLK, BLK), jnp.float32),
            pltpu.VMEM((BLK, BLK), jnp.float32),
            pltpu.VMEM((EMBEDDING_DIM, BLK), jnp.float32),
            pltpu.VMEM((EMBEDDING_DIM, BLK), jnp.float32),
            pltpu.SemaphoreType.DMA,
            pltpu.SemaphoreType.DMA,
            pltpu.SemaphoreType.DMA,
            pltpu.SemaphoreType.DMA,
            pltpu.SemaphoreType.DMA,
            pltpu.SemaphoreType.DMA,
        ],
    )
    return f(comb, w2)


@jax.jit
def _embed(token_ids, weight):
    tok_t = token_ids.astype(jnp.int32).T
    comb = jnp.stack([tok_t >> 1, (tok_t & 1) * EMBEDDING_DIM], axis=1)
    w2 = _pack(weight.T)
    out3 = _gather(comb, w2)
    return out3.transpose(2, 0, 1)


def kernel(token_ids, weight):
    return _embed(token_ids, weight)


# speedup vs baseline: 1.3544x; 1.3417x over previous
import jax
import jax.numpy as jnp
from jax import lax
from jax.experimental import pallas as pl
from jax.experimental.pallas import tpu as pltpu
from jax.experimental.pallas import tpu_sc as plsc

NUM_EMBEDDINGS = 1000000
EMBEDDING_DIM = 64
BATCH = 16384
HIST_LEN = 50

NC = 2
NS = 16
NW = NC * NS

L = 16
BLK = 128
NPAIR = NUM_EMBEDDINGS // 2
UNITS_PER_W = HIST_LEN * (BATCH // BLK) // NW
GATHER_PAIRS = UNITS_PER_W // 2


def _iota16():
    return lax.iota(jnp.int32, L)


def _gather_body(comb_hbm, w2_hbm, out_hbm, i0, i1, g0, g1, t0, t1,
                 sin0, sin1, sg0, sg1, so0, so1):
    wid = lax.axis_index("s") * NC + lax.axis_index("c")
    ub = wid * UNITS_PER_W

    def hu(uid):
        return lax.shift_right_logical(uid, 7), lax.bitwise_and(uid, 127)

    def in_copy(uid, i_ref, sem):
        h, u = hu(uid)
        return pltpu.make_async_copy(
            comb_hbm.at[h, :, pl.ds(u * BLK, BLK)], i_ref, sem)

    def g_copy(i_ref, g_ref, sem):
        return pltpu.make_async_copy(w2_hbm.at[i_ref.at[0]], g_ref, sem)

    def out_copy(uid, t_ref, sem):
        h, u = hu(uid)
        return pltpu.make_async_copy(
            t_ref, out_hbm.at[h, :, pl.ds(u * BLK, BLK)], sem)

    def transpose(i_ref, g_ref, t_ref):
        for m in range(BLK // L):
            row_vec = _iota16() + (m * L)
            b64 = i_ref[1, pl.ds(m * L, L)]
            for c0 in range(0, EMBEDDING_DIM, 8):
                vals = [plsc.load_gather(g_ref, [row_vec, b64 + c])
                        for c in range(c0, c0 + 8)]
                for i, c in enumerate(range(c0, c0 + 8)):
                    t_ref[c, pl.ds(m * L, L)] = vals[i]

    in_copy(ub, i0, sin0).start()
    in_copy(ub + 1, i1, sin1).start()
    in_copy(ub, i0, sin0).wait()
    g_copy(i0, g0, sg0).start()
    in_copy(ub + 1, i1, sin1).wait()

    def itr(k, carry):
        u0 = ub + 2 * k
        g_copy(i1, g1, sg1).start()

        g_copy(i0, g0, sg0).wait()

        @pl.when(k > 0)
        def _():
            out_copy(u0 - 2, t0, so0).wait()
        transpose(i0, g0, t0)
        out_copy(u0, t0, so0).start()

        @pl.when(k < GATHER_PAIRS - 1)
        def _():
            in_copy(u0 + 2, i0, sin0).start()
            in_copy(u0 + 2, i0, sin0).wait()
            g_copy(i0, g0, sg0).start()

        g_copy(i1, g1, sg1).wait()

        @pl.when(k > 0)
        def _():
            out_copy(u0 - 1, t1, so1).wait()
        transpose(i1, g1, t1)
        out_copy(u0 + 1, t1, so1).start()

        @pl.when(k < GATHER_PAIRS - 1)
        def _():
            in_copy(u0 + 3, i1, sin1).start()
            in_copy(u0 + 3, i1, sin1).wait()
        return carry

    lax.fori_loop(0, GATHER_PAIRS, itr, 0)
    out_copy(ub + UNITS_PER_W - 2, t0, so0).wait()
    out_copy(ub + UNITS_PER_W - 1, t1, so1).wait()


def _gather(comb, w2):
    mesh = plsc.VectorSubcoreMesh(core_axis_name="c", subcore_axis_name="s")
    f = pl.kernel(
        _gather_body,
        out_type=jax.ShapeDtypeStruct((HIST_LEN, EMBEDDING_DIM, BATCH),
                                      jnp.float32),
        mesh=mesh,
        compiler_params=pltpu.CompilerParams(needs_layout_passes=False),
        scratch_types=[
            pltpu.VMEM((2, BLK), jnp.int32),
            pltpu.VMEM((2, BLK), jnp.int32),
            pltpu.VMEM((BLK, BLK), jnp.float32),
            pltpu.VMEM((BLK, BLK), jnp.float32),
            pltpu.VMEM((EMBEDDING_DIM, BLK), jnp.float32),
            pltpu.VMEM((EMBEDDING_DIM, BLK), jnp.float32),
            pltpu.SemaphoreType.DMA,
            pltpu.SemaphoreType.DMA,
            pltpu.SemaphoreType.DMA,
            pltpu.SemaphoreType.DMA,
            pltpu.SemaphoreType.DMA,
            pltpu.SemaphoreType.DMA,
        ],
    )
    return f(comb, w2)


@jax.jit
def _embed(token_ids, weight):
    tok_t = token_ids.astype(jnp.int32).T
    comb = jnp.stack([tok_t >> 1, (tok_t & 1) * EMBEDDING_DIM], axis=1)
    w2 = weight.reshape(NPAIR, 2 * EMBEDDING_DIM)
    out3 = _gather(comb, w2)
    return out3.transpose(2, 0, 1)


def kernel(token_ids, weight):
    return _embed(token_ids, weight)
